# baseline (speedup 1.0000x reference)
import jax
import jax.numpy as jnp
from jax.experimental import pallas as pl

_B, _H, _U, _LK = 4, 16, 64, 4096
_BH = _B * _H
_R = 32


def _mask_kernel(idx_ref, out_ref):
    idx = idx_ref[...].reshape(_R, _U, 1)
    cols = jax.lax.broadcasted_iota(jnp.int32, (_R, _U, _LK), 2)
    out_ref[...] = (cols > idx).astype(jnp.int8)


def kernel(index, scores):
    del scores
    idx3 = index.reshape(_BH, 1, _U)
    out = pl.pallas_call(
        _mask_kernel,
        grid=(_BH // _R,),
        in_specs=[pl.BlockSpec((_R, 1, _U), lambda i: (i, 0, 0))],
        out_specs=pl.BlockSpec((_R, _U, _LK), lambda i: (i, 0, 0)),
        out_shape=jax.ShapeDtypeStruct((_BH, _U, _LK), jnp.int8),
    )(idx3)
    return out.reshape(_B, _H, _U, _LK).astype(jnp.bool_)


# speedup vs baseline: 1.0372x; 1.0372x over previous
import jax
import jax.numpy as jnp
from jax.experimental import pallas as pl

_B, _H, _U, _LK = 4, 16, 64, 4096
_BH = _B * _H
_R = 16


def _mask_kernel(idx_ref, out_ref):
    idx = idx_ref[...].reshape(_R, _U, 1)
    cols = jax.lax.broadcasted_iota(jnp.int32, (_R, _U, _LK), 2)
    out_ref[...] = (cols > idx).astype(jnp.int8)


def kernel(index, scores):
    del scores
    idx3 = index.reshape(_BH, 1, _U)
    out = pl.pallas_call(
        _mask_kernel,
        grid=(_BH // _R,),
        in_specs=[pl.BlockSpec((_R, 1, _U), lambda i: (i, 0, 0))],
        out_specs=pl.BlockSpec((_R, _U, _LK), lambda i: (i, 0, 0)),
        out_shape=jax.ShapeDtypeStruct((_BH, _U, _LK), jnp.int8),
    )(idx3)
    return out.reshape(_B, _H, _U, _LK).astype(jnp.bool_)
